# initial kernel scaffold (unmeasured)
import jax
import jax.numpy as jnp
from jax import lax
from jax.experimental import pallas as pl
from jax.experimental.pallas import tpu as pltpu


def kernel(
    x,
):
    def body(*refs):
        pass

    out_shape = jax.ShapeDtypeStruct(..., jnp.float32)
    return pl.pallas_call(body, out_shape=out_shape)(...)


# baseline (device time: 46366 ns/iter reference)
import jax
import jax.numpy as jnp
from jax import lax
from jax.experimental import pallas as pl
from jax.experimental.pallas import tpu as pltpu

N_DEV = 4


def kernel(x):
    m_per, n = x.shape
    half = m_per // 2
    out_dtype = jnp.bfloat16

    def body(x_ref, out_ref, r_send, r_recv, l_send, l_recv):
        my = lax.axis_index("i")
        left = (my - 1) % N_DEV
        right = (my + 1) % N_DEV

        barrier = pltpu.get_barrier_semaphore()
        for nbr in (left, right):
            pl.semaphore_signal(
                barrier, inc=1,
                device_id=(nbr,), device_id_type=pl.DeviceIdType.MESH,
            )
        pl.semaphore_wait(barrier, 2)

        out_ref[pl.ds(my * m_per, m_per), :] = x_ref[:, :].astype(out_dtype)

        for h in range(N_DEV - 1):
            o_r = (my - h) % N_DEV
            o_l = (my + h) % N_DEV
            rdma_r = pltpu.make_async_remote_copy(
                src_ref=out_ref.at[pl.ds(o_r * m_per, half), :],
                dst_ref=out_ref.at[pl.ds(o_r * m_per, half), :],
                send_sem=r_send.at[h],
                recv_sem=r_recv.at[h],
                device_id=(right,),
                device_id_type=pl.DeviceIdType.MESH,
            )
            rdma_l = pltpu.make_async_remote_copy(
                src_ref=out_ref.at[pl.ds(o_l * m_per + half, half), :],
                dst_ref=out_ref.at[pl.ds(o_l * m_per + half, half), :],
                send_sem=l_send.at[h],
                recv_sem=l_recv.at[h],
                device_id=(left,),
                device_id_type=pl.DeviceIdType.MESH,
            )
            rdma_r.start()
            rdma_l.start()
            rdma_r.wait()
            rdma_l.wait()

    return pl.pallas_call(
        body,
        out_shape=jax.ShapeDtypeStruct((N_DEV * m_per, n), out_dtype),
        in_specs=[pl.BlockSpec(memory_space=pltpu.VMEM)],
        out_specs=pl.BlockSpec(memory_space=pltpu.VMEM),
        scratch_shapes=[
            pltpu.SemaphoreType.DMA((N_DEV - 1,)),
            pltpu.SemaphoreType.DMA((N_DEV - 1,)),
            pltpu.SemaphoreType.DMA((N_DEV - 1,)),
            pltpu.SemaphoreType.DMA((N_DEV - 1,)),
        ],
        compiler_params=pltpu.CompilerParams(collective_id=0),
    )(x)


# device time: 44340 ns/iter; 1.0457x vs baseline; 1.0457x over previous
import jax
import jax.numpy as jnp
from jax import lax
from jax.experimental import pallas as pl
from jax.experimental.pallas import tpu as pltpu

N_DEV = 4
Q = 4


def kernel(x):
    m_per, n = x.shape
    half = m_per // 2
    sub = half // Q
    out_dtype = jnp.bfloat16

    def body(x_ref, out_ref, r_send, r_recv, l_send, l_recv):
        my = lax.axis_index("i")
        left = (my - 1) % N_DEV
        right = (my + 1) % N_DEV

        barrier = pltpu.get_barrier_semaphore()
        for nbr in (left, right):
            pl.semaphore_signal(
                barrier, inc=1,
                device_id=(nbr,), device_id_type=pl.DeviceIdType.MESH,
            )
        pl.semaphore_wait(barrier, 2)

        def r_rows(origin, q):
            return pl.ds(origin * m_per + q * sub, sub)

        def l_rows(origin, q):
            return pl.ds(origin * m_per + half + q * sub, sub)

        def send_r(h, q):
            o = (my - h) % N_DEV
            return pltpu.make_async_remote_copy(
                src_ref=out_ref.at[r_rows(o, q), :],
                dst_ref=out_ref.at[r_rows(o, q), :],
                send_sem=r_send.at[h, q],
                recv_sem=r_recv.at[h, q],
                device_id=(right,),
                device_id_type=pl.DeviceIdType.MESH,
            )

        def send_l(h, q):
            o = (my + h) % N_DEV
            return pltpu.make_async_remote_copy(
                src_ref=out_ref.at[l_rows(o, q), :],
                dst_ref=out_ref.at[l_rows(o, q), :],
                send_sem=l_send.at[h, q],
                recv_sem=l_recv.at[h, q],
                device_id=(left,),
                device_id_type=pl.DeviceIdType.MESH,
            )

        def recv_r(h, q):
            o = (my - 1 - h) % N_DEV
            return pltpu.make_async_remote_copy(
                src_ref=out_ref.at[r_rows(o, q), :],
                dst_ref=out_ref.at[r_rows(o, q), :],
                send_sem=r_send.at[h, q],
                recv_sem=r_recv.at[h, q],
                device_id=(left,),
                device_id_type=pl.DeviceIdType.MESH,
            )

        def recv_l(h, q):
            o = (my + 1 + h) % N_DEV
            return pltpu.make_async_remote_copy(
                src_ref=out_ref.at[l_rows(o, q), :],
                dst_ref=out_ref.at[l_rows(o, q), :],
                send_sem=l_send.at[h, q],
                recv_sem=l_recv.at[h, q],
                device_id=(right,),
                device_id_type=pl.DeviceIdType.MESH,
            )

        for q in range(Q):
            out_ref[r_rows(my, q), :] = x_ref[
                pl.ds(q * sub, sub), :
            ].astype(out_dtype)
            out_ref[l_rows(my, q), :] = x_ref[
                pl.ds(half + q * sub, sub), :
            ].astype(out_dtype)
            send_r(0, q).start()
            send_l(0, q).start()

        for h in range(1, N_DEV - 1):
            for q in range(Q):
                recv_r(h - 1, q).wait_recv()
                send_r(h, q).start()
                recv_l(h - 1, q).wait_recv()
                send_l(h, q).start()

        for q in range(Q):
            recv_r(N_DEV - 2, q).wait_recv()
            recv_l(N_DEV - 2, q).wait_recv()

        for h in range(N_DEV - 1):
            for q in range(Q):
                send_r(h, q).wait_send()
                send_l(h, q).wait_send()

    return pl.pallas_call(
        body,
        out_shape=jax.ShapeDtypeStruct((N_DEV * m_per, n), out_dtype),
        in_specs=[pl.BlockSpec(memory_space=pltpu.VMEM)],
        out_specs=pl.BlockSpec(memory_space=pltpu.VMEM),
        scratch_shapes=[
            pltpu.SemaphoreType.DMA((N_DEV - 1, Q)),
            pltpu.SemaphoreType.DMA((N_DEV - 1, Q)),
            pltpu.SemaphoreType.DMA((N_DEV - 1, Q)),
            pltpu.SemaphoreType.DMA((N_DEV - 1, Q)),
        ],
        compiler_params=pltpu.CompilerParams(collective_id=0),
    )(x)
